# Initial kernel scaffold; baseline (speedup 1.0000x reference)
import jax
import jax.numpy as jnp
from jax.experimental import pallas as pl


def kernel(inputs, labels):
    raise NotImplementedError("write your pallas kernel here")


# speedup vs baseline: 32.5261x; 32.5261x over previous
import dataclasses
import functools

import jax
import jax.numpy as jnp
from jax import lax
from jax.experimental import pallas as pl
from jax.experimental.pallas import tpu as pltpu
from jax.experimental.pallas import tpu_sc as plsc

N = 2_000_000
C = 128
L = 16
NC = 2
NS = 16
NW = NC * NS
PW = 62_496
NCHUNK = 6
CH = PW // NCHUNK
TAIL_BASE = PW * NW
TAIL = N - TAIL_BASE

_mesh = plsc.VectorSubcoreMesh(core_axis_name="c", subcore_axis_name="s")

_cp = pltpu.CompilerParams()
if "needs_layout_passes" in pltpu.CompilerParams.__dataclass_fields__:
    _cp = dataclasses.replace(_cp, needs_layout_passes=False)


@functools.partial(
    pl.kernel,
    out_type=jax.ShapeDtypeStruct((NW, C), jnp.float32),
    mesh=_mesh,
    compiler_params=_cp,
    scratch_types=[
        pltpu.VMEM((CH,), jnp.float32),
        pltpu.VMEM((CH,), jnp.int32),
        pltpu.VMEM((L, C), jnp.float32),
        pltpu.VMEM((C,), jnp.float32),
    ],
)
def _sc_partial_sums(x_hbm, lab_hbm, out_hbm, vbuf, lbuf, acc, obuf):
    wid = lax.axis_index("s") * NC + lax.axis_index("c")
    zero = jnp.zeros((L,), jnp.float32)
    for r in range(L):
        for cb in range(C // L):
            acc[r, pl.ds(cb * L, L)] = zero
    lane = lax.iota(jnp.int32, L)
    base = wid * PW

    def accum_block(n):
        @pl.loop(0, n, step=L)
        def _(i):
            vals = vbuf[pl.ds(i, L)]
            labs = lbuf[pl.ds(i, L)]
            plsc.addupdate_scatter(acc, [lane, labs], vals)

    for j in range(NCHUNK):
        pltpu.sync_copy(x_hbm.at[pl.ds(base + j * CH, CH)], vbuf)
        pltpu.sync_copy(lab_hbm.at[pl.ds(base + j * CH, CH)], lbuf)
        accum_block(CH)

    @pl.when(wid == 0)
    def _():
        pltpu.sync_copy(x_hbm.at[pl.ds(TAIL_BASE, TAIL)], vbuf.at[pl.ds(0, TAIL)])
        pltpu.sync_copy(lab_hbm.at[pl.ds(TAIL_BASE, TAIL)], lbuf.at[pl.ds(0, TAIL)])
        accum_block(TAIL)

    for cb in range(C // L):
        s = acc[0, pl.ds(cb * L, L)]
        for r in range(1, L):
            s = s + acc[r, pl.ds(cb * L, L)]
        obuf[pl.ds(cb * L, L)] = s
    pltpu.sync_copy(obuf, out_hbm.at[wid])


def _tc_reduce(partials):
    def body(in_ref, out_ref):
        out_ref[...] = jnp.sum(in_ref[...], axis=0, keepdims=True)

    return pl.pallas_call(
        body,
        out_shape=jax.ShapeDtypeStruct((1, C), jnp.float32),
    )(partials)


def kernel(inputs, labels):
    partials = _sc_partial_sums(inputs, labels)
    return _tc_reduce(partials).reshape((C,))


# speedup vs baseline: 37.6231x; 1.1567x over previous
import dataclasses
import functools

import jax
import jax.numpy as jnp
from jax import lax
from jax.experimental import pallas as pl
from jax.experimental.pallas import tpu as pltpu
from jax.experimental.pallas import tpu_sc as plsc

N = 2_000_000
C = 128
L = 16
NC = 2
NS = 16
NW = NC * NS
PW = 62_496
NCHUNK = 3
CH = PW // NCHUNK
UNROLL = 6
TAIL_BASE = PW * NW
TAIL = N - TAIL_BASE

_mesh = plsc.VectorSubcoreMesh(core_axis_name="c", subcore_axis_name="s")

_cp = pltpu.CompilerParams()
if "needs_layout_passes" in pltpu.CompilerParams.__dataclass_fields__:
    _cp = dataclasses.replace(_cp, needs_layout_passes=False)


@functools.partial(
    pl.kernel,
    out_type=jax.ShapeDtypeStruct((NW, C), jnp.float32),
    mesh=_mesh,
    compiler_params=_cp,
    scratch_types=[
        pltpu.VMEM((CH,), jnp.float32),
        pltpu.VMEM((CH,), jnp.float32),
        pltpu.VMEM((CH,), jnp.int32),
        pltpu.VMEM((CH,), jnp.int32),
        pltpu.VMEM((L, C), jnp.float32),
        pltpu.VMEM((C,), jnp.float32),
        pltpu.SemaphoreType.DMA,
        pltpu.SemaphoreType.DMA,
        pltpu.SemaphoreType.DMA,
        pltpu.SemaphoreType.DMA,
    ],
)
def _sc_partial_sums(x_hbm, lab_hbm, out_hbm, vbuf0, vbuf1, lbuf0, lbuf1,
                     acc, obuf, sv0, sv1, sl0, sl1):
    wid = lax.axis_index("s") * NC + lax.axis_index("c")
    zero = jnp.zeros((L,), jnp.float32)
    for r in range(L):
        for cb in range(C // L):
            acc[r, pl.ds(cb * L, L)] = zero
    lane = lax.iota(jnp.int32, L)
    base = wid * PW

    bufs = [(vbuf0, lbuf0, sv0, sl0), (vbuf1, lbuf1, sv1, sl1)]

    def start(j, b):
        vb, lb, sv, sl = bufs[b]
        hv = pltpu.async_copy(x_hbm.at[pl.ds(base + j * CH, CH)], vb, sv)
        hl = pltpu.async_copy(lab_hbm.at[pl.ds(base + j * CH, CH)], lb, sl)
        return hv, hl

    def accum_block(vb, lb, nelem, unroll):
        @pl.loop(0, nelem, step=L * unroll)
        def _(i):
            for u in range(unroll):
                off = i + u * L
                plsc.addupdate_scatter(
                    acc, [lane, lb[pl.ds(off, L)]], vb[pl.ds(off, L)])

    pending = start(0, 0)
    for j in range(NCHUNK):
        nxt = start(j + 1, (j + 1) % 2) if j + 1 < NCHUNK else None
        pending[0].wait()
        pending[1].wait()
        vb, lb, _, _ = bufs[j % 2]
        accum_block(vb, lb, CH, UNROLL)
        pending = nxt

    @pl.when(wid == 0)
    def _():
        pltpu.sync_copy(x_hbm.at[pl.ds(TAIL_BASE, TAIL)], vbuf0.at[pl.ds(0, TAIL)])
        pltpu.sync_copy(lab_hbm.at[pl.ds(TAIL_BASE, TAIL)], lbuf0.at[pl.ds(0, TAIL)])
        accum_block(vbuf0, lbuf0, TAIL, 1)

    for cb in range(C // L):
        s = acc[0, pl.ds(cb * L, L)]
        for r in range(1, L):
            s = s + acc[r, pl.ds(cb * L, L)]
        obuf[pl.ds(cb * L, L)] = s
    pltpu.sync_copy(obuf, out_hbm.at[wid])


def _tc_reduce(partials):
    def body(in_ref, out_ref):
        out_ref[...] = jnp.sum(in_ref[...], axis=0, keepdims=True)

    return pl.pallas_call(
        body,
        out_shape=jax.ShapeDtypeStruct((1, C), jnp.float32),
    )(partials)


def kernel(inputs, labels):
    partials = _sc_partial_sums(inputs, labels)
    return _tc_reduce(partials).reshape((C,))


# speedup vs baseline: 57.2509x; 1.5217x over previous
import dataclasses
import functools

import jax
import jax.numpy as jnp
from jax import lax
from jax.experimental import pallas as pl
from jax.experimental.pallas import tpu as pltpu
from jax.experimental.pallas import tpu_sc as plsc

N = 2_000_000
C = 128
L = 16
NC = 2
NS = 16
NW = NC * NS
PW = 62_496
NCHUNK = 3
CH = PW // NCHUNK
UNROLL = 6
TAIL_BASE = PW * NW
TAIL = N - TAIL_BASE

_mesh = plsc.VectorSubcoreMesh(core_axis_name="c", subcore_axis_name="s")

_cp = pltpu.CompilerParams()
if "needs_layout_passes" in pltpu.CompilerParams.__dataclass_fields__:
    _cp = dataclasses.replace(_cp, needs_layout_passes=False)


@functools.partial(
    pl.kernel,
    out_type=jax.ShapeDtypeStruct((NW, C), jnp.float32),
    mesh=_mesh,
    compiler_params=_cp,
    scratch_types=[
        pltpu.VMEM((CH,), jnp.float32),
        pltpu.VMEM((CH,), jnp.float32),
        pltpu.VMEM((CH,), jnp.int32),
        pltpu.VMEM((CH,), jnp.int32),
        pltpu.VMEM((L, C), jnp.float32),
        pltpu.VMEM((C,), jnp.float32),
        pltpu.SemaphoreType.DMA,
        pltpu.SemaphoreType.DMA,
        pltpu.SemaphoreType.DMA,
        pltpu.SemaphoreType.DMA,
    ],
)
def _sc_partial_sums(x_hbm, lab_hbm, out_hbm, vbuf0, vbuf1, lbuf0, lbuf1,
                     acc, obuf, sv0, sv1, sl0, sl1):
    wid = lax.axis_index("s") * NC + lax.axis_index("c")
    zero = jnp.zeros((L,), jnp.float32)
    for r in range(L):
        for cb in range(C // L):
            acc[r, pl.ds(cb * L, L)] = zero
    lane = lax.iota(jnp.int32, L)
    base = wid * PW

    bufs = [(vbuf0, lbuf0, sv0, sl0), (vbuf1, lbuf1, sv1, sl1)]

    def start(j, b):
        vb, lb, sv, sl = bufs[b]
        hv = pltpu.async_copy(x_hbm.at[pl.ds(base + j * CH, CH)], vb, sv)
        hl = pltpu.async_copy(lab_hbm.at[pl.ds(base + j * CH, CH)], lb, sl)
        return hv, hl

    def accum_block(vb, lb, nelem, unroll):
        @plsc.parallel_loop(0, nelem, step=L, unroll=unroll)
        def _(i):
            plsc.addupdate_scatter(
                acc, [lane, lb[pl.ds(i, L)]], vb[pl.ds(i, L)])

    pending = start(0, 0)
    for j in range(NCHUNK):
        nxt = start(j + 1, (j + 1) % 2) if j + 1 < NCHUNK else None
        pending[0].wait()
        pending[1].wait()
        vb, lb, _, _ = bufs[j % 2]
        accum_block(vb, lb, CH, UNROLL)
        pending = nxt

    @pl.when(wid == 0)
    def _():
        pltpu.sync_copy(x_hbm.at[pl.ds(TAIL_BASE, TAIL)], vbuf0.at[pl.ds(0, TAIL)])
        pltpu.sync_copy(lab_hbm.at[pl.ds(TAIL_BASE, TAIL)], lbuf0.at[pl.ds(0, TAIL)])
        accum_block(vbuf0, lbuf0, TAIL, 1)

    for cb in range(C // L):
        s = acc[0, pl.ds(cb * L, L)]
        for r in range(1, L):
            s = s + acc[r, pl.ds(cb * L, L)]
        obuf[pl.ds(cb * L, L)] = s
    pltpu.sync_copy(obuf, out_hbm.at[wid])


def _tc_reduce(partials):
    def body(in_ref, out_ref):
        out_ref[...] = jnp.sum(in_ref[...], axis=0, keepdims=True)

    return pl.pallas_call(
        body,
        out_shape=jax.ShapeDtypeStruct((1, C), jnp.float32),
    )(partials)


def kernel(inputs, labels):
    partials = _sc_partial_sums(inputs, labels)
    return _tc_reduce(partials).reshape((C,))


# speedup vs baseline: 58.5944x; 1.0235x over previous
import dataclasses
import functools

import jax
import jax.numpy as jnp
from jax import lax
from jax.experimental import pallas as pl
from jax.experimental.pallas import tpu as pltpu
from jax.experimental.pallas import tpu_sc as plsc

N = 2_000_000
C = 128
L = 16
NC = 2
NS = 16
NW = NC * NS
PW = 62_496
NCHUNK = 6
NBUF = 3
CH = PW // NCHUNK
UNROLL = 7
TAIL_BASE = PW * NW
TAIL = N - TAIL_BASE

_mesh = plsc.VectorSubcoreMesh(core_axis_name="c", subcore_axis_name="s")

_cp = pltpu.CompilerParams()
if "needs_layout_passes" in pltpu.CompilerParams.__dataclass_fields__:
    _cp = dataclasses.replace(_cp, needs_layout_passes=False)


@functools.partial(
    pl.kernel,
    out_type=jax.ShapeDtypeStruct((NW, C), jnp.float32),
    mesh=_mesh,
    compiler_params=_cp,
    scratch_types=(
        [pltpu.VMEM((CH,), jnp.float32) for _ in range(NBUF)]
        + [pltpu.VMEM((CH,), jnp.int32) for _ in range(NBUF)]
        + [
            pltpu.VMEM((L, C), jnp.float32),
            pltpu.VMEM((C,), jnp.float32),
        ]
        + [pltpu.SemaphoreType.DMA for _ in range(2 * NBUF)]
    ),
)
def _sc_partial_sums(x_hbm, lab_hbm, out_hbm, *refs):
    vbufs = refs[:NBUF]
    lbufs = refs[NBUF:2 * NBUF]
    acc, obuf = refs[2 * NBUF], refs[2 * NBUF + 1]
    svs = refs[2 * NBUF + 2:2 * NBUF + 2 + NBUF]
    sls = refs[2 * NBUF + 2 + NBUF:]
    wid = lax.axis_index("s") * NC + lax.axis_index("c")
    zero = jnp.zeros((L,), jnp.float32)
    for r in range(L):
        for cb in range(C // L):
            acc[r, pl.ds(cb * L, L)] = zero
    lane = lax.iota(jnp.int32, L)
    base = wid * PW

    def start(j, b):
        hv = pltpu.async_copy(x_hbm.at[pl.ds(base + j * CH, CH)], vbufs[b], svs[b])
        hl = pltpu.async_copy(lab_hbm.at[pl.ds(base + j * CH, CH)], lbufs[b], sls[b])
        return hv, hl

    def accum_block(vb, lb, nelem, unroll):
        @plsc.parallel_loop(0, nelem, step=L, unroll=unroll)
        def _(i):
            plsc.addupdate_scatter(
                acc, [lane, lb[pl.ds(i, L)]], vb[pl.ds(i, L)])

    pending = [start(j, j) for j in range(NBUF)]
    for j in range(NCHUNK):
        b = j % NBUF
        hv, hl = pending[b]
        hv.wait()
        hl.wait()
        accum_block(vbufs[b], lbufs[b], CH, UNROLL)
        if j + NBUF < NCHUNK:
            pending[b] = start(j + NBUF, b)

    @pl.when(wid == 0)
    def _():
        pltpu.sync_copy(x_hbm.at[pl.ds(TAIL_BASE, TAIL)], vbufs[0].at[pl.ds(0, TAIL)])
        pltpu.sync_copy(lab_hbm.at[pl.ds(TAIL_BASE, TAIL)], lbufs[0].at[pl.ds(0, TAIL)])
        accum_block(vbufs[0], lbufs[0], TAIL, 1)

    for cb in range(C // L):
        s = acc[0, pl.ds(cb * L, L)]
        for r in range(1, L):
            s = s + acc[r, pl.ds(cb * L, L)]
        obuf[pl.ds(cb * L, L)] = s
    pltpu.sync_copy(obuf, out_hbm.at[wid])


def _tc_reduce(partials):
    def body(in_ref, out_ref):
        out_ref[...] = jnp.sum(in_ref[...], axis=0, keepdims=True)

    return pl.pallas_call(
        body,
        out_shape=jax.ShapeDtypeStruct((1, C), jnp.float32),
    )(partials)


def kernel(inputs, labels):
    partials = _sc_partial_sums(inputs, labels)
    return _tc_reduce(partials).reshape((C,))


# speedup vs baseline: 60.9899x; 1.0409x over previous
import dataclasses
import functools

import jax
import jax.numpy as jnp
from jax import lax
from jax.experimental import pallas as pl
from jax.experimental.pallas import tpu as pltpu
from jax.experimental.pallas import tpu_sc as plsc

N = 2_000_000
C = 128
L = 16
NC = 2
NS = 16
NW = NC * NS
PW = 62_496
NCHUNK = 18
NBUF = 3
NGROUP = NCHUNK // NBUF
CH = PW // NCHUNK
UNROLL = 7
TAIL_BASE = PW * NW
TAIL = N - TAIL_BASE

_mesh = plsc.VectorSubcoreMesh(core_axis_name="c", subcore_axis_name="s")

_cp = pltpu.CompilerParams()
if "needs_layout_passes" in pltpu.CompilerParams.__dataclass_fields__:
    _cp = dataclasses.replace(_cp, needs_layout_passes=False)


@functools.partial(
    pl.kernel,
    out_type=jax.ShapeDtypeStruct((NW, C), jnp.float32),
    mesh=_mesh,
    compiler_params=_cp,
    scratch_types=(
        [pltpu.VMEM((CH,), jnp.float32) for _ in range(NBUF)]
        + [pltpu.VMEM((CH,), jnp.int32) for _ in range(NBUF)]
        + [
            pltpu.VMEM((L, C), jnp.float32),
            pltpu.VMEM((C,), jnp.float32),
        ]
        + [pltpu.SemaphoreType.DMA for _ in range(2 * NBUF)]
    ),
)
def _sc_partial_sums(x_hbm, lab_hbm, out_hbm, *refs):
    vbufs = refs[:NBUF]
    lbufs = refs[NBUF:2 * NBUF]
    acc, obuf = refs[2 * NBUF], refs[2 * NBUF + 1]
    svs = refs[2 * NBUF + 2:2 * NBUF + 2 + NBUF]
    sls = refs[2 * NBUF + 2 + NBUF:]
    wid = lax.axis_index("s") * NC + lax.axis_index("c")
    zero = jnp.zeros((L,), jnp.float32)
    for r in range(L):
        for cb in range(C // L):
            acc[r, pl.ds(cb * L, L)] = zero
    lane = lax.iota(jnp.int32, L)
    base = wid * PW

    def start(j, b):
        pltpu.async_copy(x_hbm.at[pl.ds(base + j * CH, CH)], vbufs[b], svs[b])
        pltpu.async_copy(lab_hbm.at[pl.ds(base + j * CH, CH)], lbufs[b], sls[b])

    def wait(j, b):
        pltpu.make_async_copy(
            x_hbm.at[pl.ds(base + j * CH, CH)], vbufs[b], svs[b]).wait()
        pltpu.make_async_copy(
            lab_hbm.at[pl.ds(base + j * CH, CH)], lbufs[b], sls[b]).wait()

    def accum_block(vb, lb, nelem, unroll):
        @plsc.parallel_loop(0, nelem, step=L, unroll=unroll)
        def _(i):
            plsc.addupdate_scatter(
                acc, [lane, lb[pl.ds(i, L)]], vb[pl.ds(i, L)])

    for b in range(NBUF):
        start(b, b)

    @pl.loop(0, NGROUP - 1)
    def _(g):
        j0 = g * NBUF
        for b in range(NBUF):
            wait(j0 + b, b)
            accum_block(vbufs[b], lbufs[b], CH, UNROLL)
            start(j0 + NBUF + b, b)

    for b in range(NBUF):
        wait((NGROUP - 1) * NBUF + b, b)
        accum_block(vbufs[b], lbufs[b], CH, UNROLL)

    @pl.when(wid == 0)
    def _():
        pltpu.sync_copy(x_hbm.at[pl.ds(TAIL_BASE, TAIL)], vbufs[0].at[pl.ds(0, TAIL)])
        pltpu.sync_copy(lab_hbm.at[pl.ds(TAIL_BASE, TAIL)], lbufs[0].at[pl.ds(0, TAIL)])
        accum_block(vbufs[0], lbufs[0], TAIL, 1)

    for cb in range(C // L):
        s = acc[0, pl.ds(cb * L, L)]
        for r in range(1, L):
            s = s + acc[r, pl.ds(cb * L, L)]
        obuf[pl.ds(cb * L, L)] = s
    pltpu.sync_copy(obuf, out_hbm.at[wid])


def _tc_reduce(partials):
    def body(in_ref, out_ref):
        out_ref[...] = jnp.sum(in_ref[...], axis=0, keepdims=True)

    return pl.pallas_call(
        body,
        out_shape=jax.ShapeDtypeStruct((1, C), jnp.float32),
    )(partials)


def kernel(inputs, labels):
    partials = _sc_partial_sums(inputs, labels)
    return _tc_reduce(partials).reshape((C,))
